# baseline (speedup 1.0000x reference)
import functools

import jax
import jax.numpy as jnp
from jax import lax
from jax.experimental import pallas as pl
from jax.experimental.pallas import tpu as pltpu
from jax.experimental.pallas import tpu_sc as plsc

N_NUM = 40000
D_NUM = 10000
NT = 50000
NE = 800000
ES = 64
REG = 0.01
B = 16384

NC = 2
NS = 16
L = 16

C = 128
G = 4
SUP = C * G
EPT = NE // NS
NSUP = 98
NB = 6
LA = 4
CHK = NSUP * G
EPT_PAD = NSUP * SUP
ROWS_A = 3128
ROWS_B = NT - 15 * ROWS_A
PPT = B // (NC * NS)
PCH = PPT // C


def _spmm_body(x_ref, ed_ref, edval_ref, zeros_ref, y_ref,
               acc, E3, V3, R, dstB, gsem, ssem, esem, vsem):
    c = lax.axis_index("c")
    s = lax.axis_index("s")

    r0 = s * ROWS_A

    @pl.when(s < NS - 1)
    def _():
        pltpu.sync_copy(zeros_ref.at[pl.ds(r0, ROWS_A)],
                        acc.at[pl.ds(r0, ROWS_A)])

    @pl.when(s == NS - 1)
    def _():
        pltpu.sync_copy(zeros_ref.at[pl.ds(r0, ROWS_B)],
                        acc.at[pl.ds(r0, ROWS_B)])

    plsc.subcore_barrier()

    def load_edges(t):
        slot = lax.rem(t, 3)
        pltpu.make_async_copy(ed_ref.at[c, s, t], E3.at[slot],
                              esem.at[slot]).start()
        pltpu.make_async_copy(edval_ref.at[s, t], V3.at[slot],
                              vsem.at[slot]).start()

    def wait_edges(t):
        slot = lax.rem(t, 3)
        pltpu.make_async_copy(ed_ref.at[c, s, t], E3.at[slot],
                              esem.at[slot]).wait()
        pltpu.make_async_copy(edval_ref.at[s, t], V3.at[slot],
                              vsem.at[slot]).wait()

    def fire_gather(q):
        t = q // G
        j = lax.rem(q, G)
        b = lax.rem(q, NB)
        slot = lax.rem(t, 3)
        pltpu.make_async_copy(x_ref.at[E3.at[slot, j, 0]], R.at[b],
                              gsem.at[b]).start()

    def drain_scatter(q):
        b = lax.rem(q, NB)
        pltpu.make_async_copy(R.at[b], acc.at[dstB.at[b]],
                              ssem.at[b]).wait()

    load_edges(0)
    load_edges(1)
    wait_edges(0)
    for q0 in range(LA):
        fire_gather(q0)

    def body(q, _):
        t = q // G
        j = lax.rem(q, G)
        b = lax.rem(q, NB)
        slot = lax.rem(t, 3)

        @pl.when((j == 0) & (t + 1 < NSUP))
        def _():
            wait_edges(t + 1)

            @pl.when(t + 2 < NSUP)
            def _():
                load_edges(t + 2)

        @pl.when(q >= 2)
        def _():
            drain_scatter(q - 2)

        @pl.when(q + LA < CHK)
        def _():
            fire_gather(q + LA)

        pltpu.make_async_copy(x_ref.at[E3.at[slot, j, 0]], R.at[b],
                              gsem.at[b]).wait()
        for k2 in range(C // L):
            sl = pl.ds(k2 * L, L)
            dstB[b, sl] = E3[slot, j, 1, sl]

        def mul_body(i, _):
            vv = V3[slot, j, pl.ds(i * L, L)]
            for u in range(L):
                r = i * L + u
                sv = vv[u]
                R[b, r, pl.ds(0, L)] = R[b, r, pl.ds(0, L)] * sv
                R[b, r, pl.ds(L, L)] = R[b, r, pl.ds(L, L)] * sv
            return 0
        lax.fori_loop(0, C // L, mul_body, 0, unroll=False)
        pltpu.make_async_copy(R.at[b], acc.at[dstB.at[b]],
                              ssem.at[b]).start(add=True)
        return 0

    lax.fori_loop(0, CHK, body, 0, unroll=False)

    drain_scatter(CHK - 2)
    drain_scatter(CHK - 1)
    plsc.subcore_barrier()

    @pl.when(s < NS - 1)
    def _():
        pltpu.sync_copy(acc.at[pl.ds(r0, ROWS_A)],
                        y_ref.at[pl.ds(c * NT + r0, ROWS_A)])

    @pl.when(s == NS - 1)
    def _():
        pltpu.sync_copy(acc.at[pl.ds(r0, ROWS_B)],
                        y_ref.at[pl.ds(c * NT + r0, ROWS_B)])


@functools.lru_cache(maxsize=None)
def _make_spmm():
    mesh = plsc.VectorSubcoreMesh(core_axis_name="c", subcore_axis_name="s")
    return pl.kernel(
        _spmm_body,
        out_type=jax.ShapeDtypeStruct((NC * NT, ES // NC), jnp.float32),
        mesh=mesh,
        scratch_types=[
            pltpu.VMEM_SHARED((NT, ES // NC), jnp.float32),
            pltpu.VMEM((3, G, 2, C), jnp.int32),
            pltpu.VMEM((3, G, C), jnp.float32),
            pltpu.VMEM((NB, C, ES // NC), jnp.float32),
            pltpu.VMEM((NB, C), jnp.int32),
            pltpu.SemaphoreType.DMA((NB,)),
            pltpu.SemaphoreType.DMA((NB,)),
            pltpu.SemaphoreType.DMA((3,)),
            pltpu.SemaphoreType.DMA((3,)),
        ],
        compiler_params=pltpu.CompilerParams(use_tc_tiling_on_sc=False,
                                             needs_layout_passes=False),
        name="spmm_sc",
    )


def _final_body(tbl_ref, idx_ref, pre_ref, l2_ref,
                ib, T, S, prebuf, l2pad, tsem):
    c = lax.axis_index("c")
    s = lax.axis_index("s")
    wid = c * NS + s

    l2pad[0, pl.ds(0, L)] = jnp.zeros((L,), jnp.float32)

    for cb in range(PCH):
        base = wid * PPT + cb * C
        for r in range(4):
            pltpu.sync_copy(idx_ref.at[pl.ds(r * B + base, C)], ib.at[r])
        descs = []
        for r in range(4):
            descs.append(pltpu.async_copy(
                tbl_ref.at[ib.at[r]], T.at[r], tsem.at[r]))
        for d in descs:
            d.wait()

        W = ES // NC

        def sum_body(i, _):
            for q in range(4):
                for h in range(2):
                    so = pl.ds(i * W + h * L, L)
                    S[q, so] = (T[q, i, pl.ds(0 * W + h * L, L)]
                                + T[q, i, pl.ds(1 * W + h * L, L)]
                                + T[q, i, pl.ds(2 * W + h * L, L)]
                                + T[q, i, pl.ds(3 * W + h * L, L)])
            return 0
        lax.fori_loop(0, C, sum_body, 0, unroll=False)

        lanes = lax.iota(jnp.int32, L)

        def grp_body(g, _):
            prevec = jnp.zeros((L,), jnp.float32)
            l2v = jnp.zeros((L,), jnp.float32)
            for u in range(L):
                p = g * L + u
                nlo0 = S[0, pl.ds(p * W, L)]
                nlo1 = S[0, pl.ds(p * W + L, L)]
                nhi0 = S[1, pl.ds(p * W, L)]
                nhi1 = S[1, pl.ds(p * W + L, L)]
                dlo0 = S[2, pl.ds(p * W, L)]
                dlo1 = S[2, pl.ds(p * W + L, L)]
                dhi0 = S[3, pl.ds(p * W, L)]
                dhi1 = S[3, pl.ds(p * W + L, L)]
                dotv = (nlo0 * dlo0 + nlo1 * dlo1
                        + nhi0 * dhi0 + nhi1 * dhi1)
                dots = jnp.sum(dotv)
                prevec = jnp.where(lanes == u, dots, prevec)
                l2v = (l2v + nlo0 * nlo0 + nlo1 * nlo1 + nhi0 * nhi0
                       + nhi1 * nhi1 + dlo0 * dlo0 + dlo1 * dlo1
                       + dhi0 * dhi0 + dhi1 * dhi1)
            pre = 1.0 / (1.0 + jnp.exp(-prevec))
            prebuf[pl.ds(cb * C + g * L, L)] = pre
            l2pad[0, pl.ds(0, L)] = l2pad[0, pl.ds(0, L)] + l2v
            return 0
        lax.fori_loop(0, C // L, grp_body, 0, unroll=False)

    pltpu.sync_copy(prebuf, pre_ref.at[pl.ds(wid * PPT, PPT)])
    pltpu.sync_copy(l2pad, l2_ref.at[wid])


@functools.lru_cache(maxsize=None)
def _make_final():
    mesh = plsc.VectorSubcoreMesh(core_axis_name="c", subcore_axis_name="s")
    return pl.kernel(
        _final_body,
        out_type=(jax.ShapeDtypeStruct((B,), jnp.float32),
                  jax.ShapeDtypeStruct((NC * NS, 8, 128), jnp.float32)),
        mesh=mesh,
        scratch_types=[
            pltpu.VMEM((4, C), jnp.int32),
            pltpu.VMEM((4, C, 2 * ES), jnp.float32),
            pltpu.VMEM((4, C * (ES // NC)), jnp.float32),
            pltpu.VMEM((PPT,), jnp.float32),
            pltpu.VMEM((8, 128), jnp.float32),
            pltpu.SemaphoreType.DMA((4,)),
        ],
        compiler_params=pltpu.CompilerParams(needs_layout_passes=False),
        name="final_sc",
    )


def kernel(n, d_i, E_weight, edge_index, adj_vals):
    f32 = jnp.float32
    i32 = jnp.int32

    e_split = jnp.concatenate([E_weight[:, :ES // 2], E_weight[:, ES // 2:]],
                              axis=0)

    src = edge_index[1].reshape(NS, EPT)
    dst = edge_index[0].reshape(NS, EPT)
    vals = adj_vals.reshape(NS, EPT)
    pad = EPT_PAD - EPT
    srcp = jnp.pad(src, ((0, 0), (0, pad)))
    dstp = jnp.pad(dst, ((0, 0), (0, pad)))
    valp = jnp.pad(vals, ((0, 0), (0, pad)))
    offs = jnp.array([0, NT], i32)
    srcs = srcp[None, :, :] + offs[:, None, None]
    dst2 = jnp.broadcast_to(dstp[None], (NC, NS, EPT_PAD))
    ed = jnp.stack([srcs, dst2], axis=2)
    ed = ed.reshape(NC, NS, 2, NSUP, G, C).transpose(0, 1, 3, 4, 2, 5)
    ed_val = valp.reshape(NS, NSUP, G, C)

    zeros = jnp.zeros((NT, ES // NC), f32)

    spmm = _make_spmm()
    x1 = spmm(e_split, ed, ed_val, zeros)
    x2 = spmm(x1, ed, ed_val, zeros)
    x3 = spmm(x2, ed, ed_val, zeros)

    tbl = jnp.concatenate([e_split, x1, x2, x3], axis=1)

    idx4 = jnp.concatenate(
        [n, n + NT, N_NUM + d_i, NT + N_NUM + d_i]).astype(i32)

    final = _make_final()
    pre, l2p = final(tbl, idx4)
    l2 = (REG / B) * jnp.sum(l2p[:, 0, :L])
    return pre, l2.reshape(1)


# speedup vs baseline: 1.8216x; 1.8216x over previous
import functools

import jax
import jax.numpy as jnp
from jax import lax
from jax.experimental import pallas as pl
from jax.experimental.pallas import tpu as pltpu
from jax.experimental.pallas import tpu_sc as plsc

N_NUM = 40000
D_NUM = 10000
NT = 50000
NE = 800000
ES = 64
REG = 0.01
B = 16384

NC = 2
NS = 16
L = 16

C = 128
G = 4
SUP = C * G
EPT = NE // NS
NSUP = 98
NB = 6
LA = 4
CHK = NSUP * G
EPT_PAD = NSUP * SUP
ROWS_A = 3128
ROWS_B = NT - 15 * ROWS_A
PPT = B // (NC * NS)
PCH = PPT // C


def _spmm_body(x_ref, ed_ref, edval_ref, zeros_ref, y_ref,
               acc, EA, EB, VA, VB, R, dstB, gsem, ssem, esem):
    c = lax.axis_index("c")
    s = lax.axis_index("s")

    r0 = s * ROWS_A

    @pl.when(s < NS - 1)
    def _():
        pltpu.sync_copy(zeros_ref.at[pl.ds(r0, ROWS_A)],
                        acc.at[pl.ds(r0, ROWS_A)])

    @pl.when(s == NS - 1)
    def _():
        pltpu.sync_copy(zeros_ref.at[pl.ds(r0, ROWS_B)],
                        acc.at[pl.ds(r0, ROWS_B)])

    plsc.subcore_barrier()

    def process_super(g_sup, E, V):
        for j in range(G):
            @pl.when(g_sup > 0)
            def _():
                pltpu.make_async_copy(R.at[j], acc.at[dstB.at[j]],
                                      ssem.at[j]).wait()
            pltpu.make_async_copy(x_ref.at[E.at[j, 0]], R.at[j],
                                  gsem.at[j]).start()
        for j in range(G):
            pltpu.make_async_copy(x_ref.at[E.at[j, 0]], R.at[j],
                                  gsem.at[j]).wait()
            for k2 in range(C // L):
                sl = pl.ds(k2 * L, L)
                dstB[j, sl] = E[j, 1, sl]

            def mul_body(i, _):
                vv = V[j, pl.ds(i * L, L)]
                for u in range(L):
                    r = i * L + u
                    sv = vv[u]
                    R[j, r, pl.ds(0, L)] = R[j, r, pl.ds(0, L)] * sv
                    R[j, r, pl.ds(L, L)] = R[j, r, pl.ds(L, L)] * sv
                return 0
            lax.fori_loop(0, C // L, mul_body, 0, unroll=False)
            pltpu.make_async_copy(R.at[j], acc.at[dstB.at[j]],
                                  ssem.at[j]).start(add=True)

    pltpu.async_copy(ed_ref.at[c, s, 0], EA, esem.at[0])
    pltpu.async_copy(edval_ref.at[s, 0], VA, esem.at[2])

    def outer(i, _):
        a = 2 * i
        pltpu.make_async_copy(ed_ref.at[c, s, a], EA, esem.at[0]).wait()
        pltpu.make_async_copy(edval_ref.at[s, a], VA, esem.at[2]).wait()
        pltpu.async_copy(ed_ref.at[c, s, a + 1], EB, esem.at[1])
        pltpu.async_copy(edval_ref.at[s, a + 1], VB, esem.at[3])
        process_super(a, EA, VA)

        @pl.when(i < NSUP // 2 - 1)
        def _():
            pltpu.async_copy(ed_ref.at[c, s, a + 2], EA, esem.at[0])
            pltpu.async_copy(edval_ref.at[s, a + 2], VA, esem.at[2])
        pltpu.make_async_copy(ed_ref.at[c, s, a + 1], EB, esem.at[1]).wait()
        pltpu.make_async_copy(edval_ref.at[s, a + 1], VB, esem.at[3]).wait()
        process_super(a + 1, EB, VB)
        return 0

    lax.fori_loop(0, NSUP // 2, outer, 0, unroll=False)

    for j in range(G):
        pltpu.make_async_copy(R.at[j], acc.at[dstB.at[j]], ssem.at[j]).wait()
    plsc.subcore_barrier()

    @pl.when(s < NS - 1)
    def _():
        pltpu.sync_copy(acc.at[pl.ds(r0, ROWS_A)],
                        y_ref.at[pl.ds(c * NT + r0, ROWS_A)])

    @pl.when(s == NS - 1)
    def _():
        pltpu.sync_copy(acc.at[pl.ds(r0, ROWS_B)],
                        y_ref.at[pl.ds(c * NT + r0, ROWS_B)])


@functools.lru_cache(maxsize=None)
def _make_spmm():
    mesh = plsc.VectorSubcoreMesh(core_axis_name="c", subcore_axis_name="s")
    return pl.kernel(
        _spmm_body,
        out_type=jax.ShapeDtypeStruct((NC * NT, ES // NC), jnp.float32),
        mesh=mesh,
        scratch_types=[
            pltpu.VMEM_SHARED((NT, ES // NC), jnp.float32),
            pltpu.VMEM((G, 2, C), jnp.int32),
            pltpu.VMEM((G, 2, C), jnp.int32),
            pltpu.VMEM((G, C), jnp.float32),
            pltpu.VMEM((G, C), jnp.float32),
            pltpu.VMEM((G, C, ES // NC), jnp.float32),
            pltpu.VMEM((G, C), jnp.int32),
            pltpu.SemaphoreType.DMA((G,)),
            pltpu.SemaphoreType.DMA((G,)),
            pltpu.SemaphoreType.DMA((4,)),
        ],
        compiler_params=pltpu.CompilerParams(use_tc_tiling_on_sc=False,
                                             needs_layout_passes=False),
        name="spmm_sc",
    )


def _final_body(tbl_ref, idx_ref, pre_ref, l2_ref,
                ib, T, S, prebuf, l2pad, tsem):
    c = lax.axis_index("c")
    s = lax.axis_index("s")
    wid = c * NS + s

    l2pad[0, pl.ds(0, L)] = jnp.zeros((L,), jnp.float32)

    for cb in range(PCH):
        base = wid * PPT + cb * C
        for r in range(4):
            pltpu.sync_copy(idx_ref.at[pl.ds(r * B + base, C)], ib.at[r])
        descs = []
        for r in range(4):
            descs.append(pltpu.async_copy(
                tbl_ref.at[ib.at[r]], T.at[r], tsem.at[r]))
        for d in descs:
            d.wait()

        W = ES // NC

        def sum_body(i, _):
            for q in range(4):
                for h in range(2):
                    so = pl.ds(i * W + h * L, L)
                    S[q, so] = (T[q, i, pl.ds(0 * W + h * L, L)]
                                + T[q, i, pl.ds(1 * W + h * L, L)]
                                + T[q, i, pl.ds(2 * W + h * L, L)]
                                + T[q, i, pl.ds(3 * W + h * L, L)])
            return 0
        lax.fori_loop(0, C, sum_body, 0, unroll=False)

        lanes = lax.iota(jnp.int32, L)

        def grp_body(g, _):
            prevec = jnp.zeros((L,), jnp.float32)
            l2v = jnp.zeros((L,), jnp.float32)
            for u in range(L):
                p = g * L + u
                nlo0 = S[0, pl.ds(p * W, L)]
                nlo1 = S[0, pl.ds(p * W + L, L)]
                nhi0 = S[1, pl.ds(p * W, L)]
                nhi1 = S[1, pl.ds(p * W + L, L)]
                dlo0 = S[2, pl.ds(p * W, L)]
                dlo1 = S[2, pl.ds(p * W + L, L)]
                dhi0 = S[3, pl.ds(p * W, L)]
                dhi1 = S[3, pl.ds(p * W + L, L)]
                dotv = (nlo0 * dlo0 + nlo1 * dlo1
                        + nhi0 * dhi0 + nhi1 * dhi1)
                dots = jnp.sum(dotv)
                prevec = jnp.where(lanes == u, dots, prevec)
                l2v = (l2v + nlo0 * nlo0 + nlo1 * nlo1 + nhi0 * nhi0
                       + nhi1 * nhi1 + dlo0 * dlo0 + dlo1 * dlo1
                       + dhi0 * dhi0 + dhi1 * dhi1)
            pre = 1.0 / (1.0 + jnp.exp(-prevec))
            prebuf[pl.ds(cb * C + g * L, L)] = pre
            l2pad[0, pl.ds(0, L)] = l2pad[0, pl.ds(0, L)] + l2v
            return 0
        lax.fori_loop(0, C // L, grp_body, 0, unroll=False)

    pltpu.sync_copy(prebuf, pre_ref.at[pl.ds(wid * PPT, PPT)])
    pltpu.sync_copy(l2pad, l2_ref.at[wid])


@functools.lru_cache(maxsize=None)
def _make_final():
    mesh = plsc.VectorSubcoreMesh(core_axis_name="c", subcore_axis_name="s")
    return pl.kernel(
        _final_body,
        out_type=(jax.ShapeDtypeStruct((B,), jnp.float32),
                  jax.ShapeDtypeStruct((NC * NS, 8, 128), jnp.float32)),
        mesh=mesh,
        scratch_types=[
            pltpu.VMEM((4, C), jnp.int32),
            pltpu.VMEM((4, C, 2 * ES), jnp.float32),
            pltpu.VMEM((4, C * (ES // NC)), jnp.float32),
            pltpu.VMEM((PPT,), jnp.float32),
            pltpu.VMEM((8, 128), jnp.float32),
            pltpu.SemaphoreType.DMA((4,)),
        ],
        compiler_params=pltpu.CompilerParams(needs_layout_passes=False),
        name="final_sc",
    )


def kernel(n, d_i, E_weight, edge_index, adj_vals):
    f32 = jnp.float32
    i32 = jnp.int32

    e_split = jnp.concatenate([E_weight[:, :ES // 2], E_weight[:, ES // 2:]],
                              axis=0)

    src = edge_index[1].reshape(NS, EPT)
    dst = edge_index[0].reshape(NS, EPT)
    vals = adj_vals.reshape(NS, EPT)
    pad = EPT_PAD - EPT
    srcp = jnp.pad(src, ((0, 0), (0, pad)))
    dstp = jnp.pad(dst, ((0, 0), (0, pad)))
    valp = jnp.pad(vals, ((0, 0), (0, pad)))
    offs = jnp.array([0, NT], i32)
    srcs = srcp[None, :, :] + offs[:, None, None]
    dst2 = jnp.broadcast_to(dstp[None], (NC, NS, EPT_PAD))
    ed = jnp.stack([srcs, dst2], axis=2)
    ed = ed.reshape(NC, NS, 2, NSUP, G, C).transpose(0, 1, 3, 4, 2, 5)
    ed_val = valp.reshape(NS, NSUP, G, C)

    zeros = jnp.zeros((NT, ES // NC), f32)

    spmm = _make_spmm()
    x1 = spmm(e_split, ed, ed_val, zeros)
    x2 = spmm(x1, ed, ed_val, zeros)
    x3 = spmm(x2, ed, ed_val, zeros)

    tbl = jnp.concatenate([e_split, x1, x2, x3], axis=1)

    idx4 = jnp.concatenate(
        [n, n + NT, N_NUM + d_i, NT + N_NUM + d_i]).astype(i32)

    final = _make_final()
    pre, l2p = final(tbl, idx4)
    l2 = (REG / B) * jnp.sum(l2p[:, 0, :L])
    return pre, l2.reshape(1)
